# baseline (speedup 1.0000x reference)
import functools

import jax
import jax.numpy as jnp
from jax import lax
from jax.experimental import pallas as pl
from jax.experimental.pallas import tpu as pltpu
from jax.experimental.pallas import tpu_sc as plsc

CH = 128


@functools.cache
def _build(N, V, D, n_ch, NC, NS):
  NW = NC * NS
  per_w = N // NW
  mesh = plsc.VectorSubcoreMesh(core_axis_name="c", subcore_axis_name="s")

  @functools.partial(
      pl.kernel,
      mesh=mesh,
      compiler_params=pltpu.CompilerParams(use_tc_tiling_on_sc=False),
      out_type=jax.ShapeDtypeStruct((N, D), jnp.float32),
      scratch_types=[
          pltpu.VMEM((n_ch, CH), jnp.int32),
          pltpu.VMEM((CH, D), jnp.float32),
          pltpu.VMEM((CH, D), jnp.float32),
          pltpu.SemaphoreType.DMA,
          pltpu.SemaphoreType.DMA,
      ],
  )
  def k(idx_hbm, table_hbm, out_hbm, idx_v, buf0, buf1, sem0, sem1):
    wid = lax.axis_index("s") * NC + lax.axis_index("c")
    base = wid * per_w
    pltpu.sync_copy(idx_hbm.at[wid], idx_v)
    bufs = (buf0, buf1)
    sems = (sem0, sem1)
    pltpu.async_copy(table_hbm.at[idx_v.at[0]], bufs[0], sems[0])

    @pl.loop(0, n_ch, step=2)
    def _(j):
      for b in range(2):
        cur = j + b
        nxt = cur + 1

        @pl.when(nxt < n_ch)
        def _():
          pltpu.async_copy(
              table_hbm.at[idx_v.at[nxt]], bufs[1 - b], sems[1 - b])

        pltpu.make_async_copy(
            table_hbm.at[idx_v.at[cur]], bufs[b], sems[b]).wait()
        pltpu.sync_copy(bufs[b], out_hbm.at[pl.ds(base + cur * CH, CH)])

  return k


def kernel(x, weight):
  B_, L_ = x.shape
  V, D = weight.shape
  N = B_ * L_
  info = plsc.get_sparse_core_info()
  NC, NS = info.num_cores, info.num_subcores
  NW = NC * NS
  per_w = N // NW
  n_ch = per_w // CH
  idx = x.reshape(NW, n_ch, CH).astype(jnp.int32)
  out = _build(N, V, D, n_ch, NC, NS)(idx, weight)
  return out.reshape(B_, L_, D)


# speedup vs baseline: 1.0092x; 1.0092x over previous
import functools

import jax
import jax.numpy as jnp
from jax import lax
from jax.experimental import pallas as pl
from jax.experimental.pallas import tpu as pltpu
from jax.experimental.pallas import tpu_sc as plsc

CHM = 128
KSUB = 2
CH = KSUB * CHM
NBUF = 4


@functools.cache
def _build(N, V, D, n_ch, NC, NS):
  NW = NC * NS
  mesh = plsc.VectorSubcoreMesh(core_axis_name="c", subcore_axis_name="s")
  n_loop = ((n_ch + NBUF - 1) // NBUF) * NBUF

  @functools.partial(
      pl.kernel,
      mesh=mesh,
      compiler_params=pltpu.CompilerParams(use_tc_tiling_on_sc=False),
      out_type=jax.ShapeDtypeStruct((N // CH, KSUB, CHM, D), jnp.float32),
      scratch_types=[
          pltpu.VMEM((n_ch, KSUB, CHM), jnp.int32),
          pltpu.VMEM((NBUF, KSUB, CHM, D), jnp.float32),
          [pltpu.SemaphoreType.DMA] * NBUF,
          [pltpu.SemaphoreType.DMA] * NBUF,
      ],
  )
  def k(idx_hbm, table_hbm, out_hbm, idx_v, bufs, gsems, wsems):
    wid = lax.axis_index("s") * NC + lax.axis_index("c")
    base = wid * n_ch
    pltpu.sync_copy(idx_hbm.at[wid], idx_v)

    def gather(c, b):
      for s in range(KSUB):
        pltpu.async_copy(
            table_hbm.at[idx_v.at[c, s]], bufs.at[b, s], gsems[b])

    def wait_gather(c, b):
      for s in range(KSUB):
        pltpu.make_async_copy(
            table_hbm.at[idx_v.at[c, s]], bufs.at[b, s], gsems[b]).wait()

    def write(c, b):
      pltpu.async_copy(bufs.at[b], out_hbm.at[base + c], wsems[b])

    def wait_write(c, b):
      pltpu.make_async_copy(bufs.at[b], out_hbm.at[base + c], wsems[b]).wait()

    gather(0, 0)
    gather(1, 1)

    @pl.loop(0, n_loop, step=NBUF)
    def _(j):
      for b in range(NBUF):
        c = j + b

        @pl.when(jnp.logical_and(c - 2 >= 0, c - 2 < n_ch))
        def _():
          wait_write(c - 2, (b - 2) % NBUF)

        @pl.when(c + 2 < n_ch)
        def _():
          gather(c + 2, (b + 2) % NBUF)

        @pl.when(c < n_ch)
        def _():
          wait_gather(c, b)
          write(c, b)

    for c in range(max(0, n_loop - 2), n_ch):
      wait_write(c, c % NBUF)

  return k


def kernel(x, weight):
  B_, L_ = x.shape
  V, D = weight.shape
  N = B_ * L_
  info = plsc.get_sparse_core_info()
  NC, NS = info.num_cores, info.num_subcores
  NW = NC * NS
  per_w = N // NW
  n_ch = per_w // CH
  idx = x.reshape(NW, n_ch, KSUB, CHM).astype(jnp.int32)
  out = _build(N, V, D, n_ch, NC, NS)(idx, weight)
  return out.reshape(B_, L_, D)
